# baseline (speedup 1.0000x reference)
import functools

import jax
import jax.numpy as jnp
from jax import lax
from jax.experimental import pallas as pl
from jax.experimental.pallas import tpu as pltpu
from jax.experimental.pallas import tpu_sc as plsc

N = 100000
E = 1600000
NC = 2
NS = 16
CH = 100
ER = E // CH
KS = 8
RPT = ER // NS
NSTEP = RPT // KS
RB = 1000
NBLK = N // RB
ZB = 128
NZB = N // ZB
ZTAIL = N - NZB * ZB
B = 4000
NB = N // B
F32 = jnp.float32

_mesh = plsc.VectorSubcoreMesh(core_axis_name="c", subcore_axis_name="s")


def _zero_acc(zbuf, acc, s):
    zero = jnp.zeros((16,), F32)

    @pl.loop(0, ZB)
    def _(i):
        zbuf[i, :] = zero

    @pl.loop(s, NZB, step=NS)
    def _(j):
        pltpu.sync_copy(zbuf, acc.at[pl.ds(j * ZB, ZB)])

    @pl.when(s == 0)
    def _():
        pltpu.sync_copy(zbuf.at[pl.ds(0, ZTAIL)],
                        acc.at[pl.ds(NZB * ZB, ZTAIL)])


@functools.partial(
    pl.kernel,
    out_type=jax.ShapeDtypeStruct((NC, N, 16), F32),
    mesh=_mesh,
    compiler_params=pltpu.CompilerParams(use_tc_tiling_on_sc=False),
    scratch_types=[
        pltpu.VMEM((KS, CH), jnp.int32),
        pltpu.VMEM((CH, 16), F32),
        pltpu.VMEM((ZB, 16), F32),
        pltpu.VMEM_SHARED((N, 16), F32),
        pltpu.SemaphoreType.DMA,
    ],
)
def _deg_kernel(src2_hbm, dst2_hbm, out_hbm, idx, ones_r, zbuf, acc, sem):
    c = lax.axis_index("c")
    s = lax.axis_index("s")
    one = jnp.ones((16,), F32)

    @pl.loop(0, CH)
    def _(i):
        ones_r[i, :] = one

    _zero_acc(zbuf, acc, s)

    plsc.subcore_barrier()

    row0 = s * RPT

    def count_pass(e2_hbm):
        @pl.loop(0, NSTEP)
        def _(j):
            r = pl.multiple_of(row0 + j * KS, 8)
            pltpu.sync_copy(e2_hbm.at[pl.ds(r, KS)], idx)
            descs = [pltpu.async_copy(ones_r, acc.at[idx.at[k]], sem,
                                      add=True) for k in range(KS)]
            for d in descs:
                d.wait()

    @pl.when(c == 0)
    def _():
        count_pass(src2_hbm)

    @pl.when(c == 1)
    def _():
        count_pass(dst2_hbm)

    plsc.subcore_barrier()

    @pl.loop(s, NBLK, step=NS)
    def _(j):
        pltpu.sync_copy(acc.at[pl.ds(j * RB, RB)],
                        out_hbm.at[c, pl.ds(j * RB, RB)])


@functools.partial(
    pl.kernel,
    out_type=[jax.ShapeDtypeStruct((N, 16), F32),
              jax.ShapeDtypeStruct((N, 16), F32)],
    mesh=_mesh,
    compiler_params=pltpu.CompilerParams(use_tc_tiling_on_sc=False),
    scratch_types=[
        pltpu.VMEM((KS, CH), jnp.int32),
        pltpu.VMEM((KS, CH), jnp.int32),
        pltpu.VMEM((KS, CH), jnp.int32),
        pltpu.VMEM((KS, CH), jnp.int32),
        pltpu.VMEM((KS * CH, 16), F32),
        pltpu.VMEM((KS * CH, 16), F32),
        pltpu.VMEM((ZB, 16), F32),
        pltpu.VMEM_SHARED((N, 16), F32),
        pltpu.SemaphoreType.DMA,
        pltpu.SemaphoreType.DMA,
        pltpu.SemaphoreType.DMA,
        pltpu.SemaphoreType.DMA,
    ],
)
def _spmm_kernel(hlo_hbm, hhi_hbm, src2_hbm, dst2_hbm, olo_hbm, ohi_hbm,
                 sidx0, didx0, sidx1, didx1, rows0, rows1, zbuf, acc,
                 gsem0, gsem1, ssem0, ssem1):
    c = lax.axis_index("c")
    s = lax.axis_index("s")

    _zero_acc(zbuf, acc, s)

    plsc.subcore_barrier()

    row0 = s * RPT
    HB = KS * CH

    def edge_pass(h_hbm):
        def load_idx(j, si, di):
            r = pl.multiple_of(row0 + j * KS, 8)
            pltpu.sync_copy(src2_hbm.at[pl.ds(r, KS)], si)
            pltpu.sync_copy(dst2_hbm.at[pl.ds(r, KS)], di)

        def fire_g(si, rb, sem):
            for k in range(KS):
                pltpu.async_copy(h_hbm.at[si.at[k]],
                                 rb.at[pl.ds(k * CH, CH)], sem)

        def fire_s(di, rb, sem):
            for k in range(KS):
                pltpu.async_copy(rb.at[pl.ds(k * CH, CH)],
                                 acc.at[di.at[k]], sem, add=True)

        def drain(sem, rb):
            pltpu.make_async_copy(h_hbm.at[pl.ds(0, HB)], rb, sem).wait()

        load_idx(0, sidx0, didx0)
        fire_g(sidx0, rows0, gsem0)

        @pl.loop(0, NSTEP // 2)
        def _(t):
            j = 2 * t
            load_idx(j + 1, sidx1, didx1)
            fire_g(sidx1, rows1, gsem1)
            drain(gsem0, rows0)
            fire_s(didx0, rows0, ssem0)
            drain(ssem0, rows0)

            @pl.when(t + 1 < NSTEP // 2)
            def _():
                load_idx(j + 2, sidx0, didx0)
                fire_g(sidx0, rows0, gsem0)

            drain(gsem1, rows1)
            fire_s(didx1, rows1, ssem1)
            drain(ssem1, rows1)

        if NSTEP % 2 == 1:
            load_idx(NSTEP - 1, sidx0, didx0)
            fire_g(sidx0, rows0, gsem0)
            drain(gsem0, rows0)
            fire_s(didx0, rows0, ssem0)
            drain(ssem0, rows0)

    @pl.when(c == 0)
    def _():
        edge_pass(hlo_hbm)

    @pl.when(c == 1)
    def _():
        edge_pass(hhi_hbm)

    plsc.subcore_barrier()

    def copy_out(o_hbm):
        @pl.loop(s, NBLK, step=NS)
        def _(j):
            pltpu.sync_copy(acc.at[pl.ds(j * RB, RB)],
                            o_hbm.at[pl.ds(j * RB, RB)])

    @pl.when(c == 0)
    def _():
        copy_out(olo_hbm)

    @pl.when(c == 1)
    def _():
        copy_out(ohi_hbm)


def _pre1_body(x_ref, cnt_ref, w_ref, lo_ref, hi_ref, si_ref, so_ref):
    so = lax.rsqrt(jnp.maximum(cnt_ref[0, :, 0:1], 1.0))
    si = lax.rsqrt(jnp.maximum(cnt_ref[1, :, 0:1], 1.0))
    z = jnp.dot(x_ref[...] * so, w_ref[...], preferred_element_type=F32)
    lo_ref[...] = z[:, :16]
    hi_ref[...] = z[:, 16:]
    si_ref[...] = si
    so_ref[...] = so


def _pre1(x, cnt, W1):
    return pl.pallas_call(
        _pre1_body,
        grid=(NB,),
        in_specs=[
            pl.BlockSpec((B, 33), lambda i: (i, 0)),
            pl.BlockSpec((NC, B, 16), lambda i: (0, i, 0)),
            pl.BlockSpec((33, 32), lambda i: (0, 0)),
        ],
        out_specs=[
            pl.BlockSpec((B, 16), lambda i: (i, 0)),
            pl.BlockSpec((B, 16), lambda i: (i, 0)),
            pl.BlockSpec((B, 1), lambda i: (i, 0)),
            pl.BlockSpec((B, 1), lambda i: (i, 0)),
        ],
        out_shape=[
            jax.ShapeDtypeStruct((N, 16), F32),
            jax.ShapeDtypeStruct((N, 16), F32),
            jax.ShapeDtypeStruct((N, 1), F32),
            jax.ShapeDtypeStruct((N, 1), F32),
        ],
    )(x, cnt, W1)


def _bn_lrelu(t, st_ref, g_ref, be_ref):
    mu = st_ref[0, :] * (1.0 / N)
    var = st_ref[1, :] * (1.0 / N) - mu * mu
    y = (t - mu[None, :]) * lax.rsqrt(var + 1e-5)[None, :]
    y = y * g_ref[...] + be_ref[...]
    return jnp.where(y >= 0, y, 0.01 * y)


def _accum_stats(t, st_ref):
    s1 = jnp.sum(t, axis=0)
    s2 = jnp.sum(t * t, axis=0)
    upd = jnp.concatenate([s1[None], s2[None], jnp.zeros((6, 32), F32)],
                          axis=0)

    @pl.when(pl.program_id(1) == 0)
    def _():
        st_ref[...] = jnp.zeros_like(st_ref)

    st_ref[...] += upd


def _mid_body(lo_ref, hi_ref, si_ref, so_ref, b_ref, g_ref, be_ref,
              w_ref, olo_ref, ohi_ref, st_ref):
    t = jnp.concatenate([lo_ref[...], hi_ref[...]], axis=1)
    t = t * si_ref[...] + b_ref[...]
    ph = pl.program_id(0)

    @pl.when(ph == 0)
    def _():
        _accum_stats(t, st_ref)

    @pl.when(ph == 1)
    def _():
        y = _bn_lrelu(t, st_ref, g_ref, be_ref)
        z = jnp.dot(y * so_ref[...], w_ref[...], preferred_element_type=F32)
        olo_ref[...] = z[:, :16]
        ohi_ref[...] = z[:, 16:]


def _mid(alo, ahi, si, so, b, g, be, Wn):
    return pl.pallas_call(
        _mid_body,
        grid=(2, NB),
        in_specs=[
            pl.BlockSpec((B, 16), lambda p, i: (i, 0)),
            pl.BlockSpec((B, 16), lambda p, i: (i, 0)),
            pl.BlockSpec((B, 1), lambda p, i: (i, 0)),
            pl.BlockSpec((B, 1), lambda p, i: (i, 0)),
            pl.BlockSpec((1, 32), lambda p, i: (0, 0)),
            pl.BlockSpec((1, 32), lambda p, i: (0, 0)),
            pl.BlockSpec((1, 32), lambda p, i: (0, 0)),
            pl.BlockSpec((32, 32), lambda p, i: (0, 0)),
        ],
        out_specs=[
            pl.BlockSpec((B, 16), lambda p, i: (i, 0)),
            pl.BlockSpec((B, 16), lambda p, i: (i, 0)),
        ],
        out_shape=[
            jax.ShapeDtypeStruct((N, 16), F32),
            jax.ShapeDtypeStruct((N, 16), F32),
        ],
        scratch_shapes=[pltpu.VMEM((8, 32), F32)],
    )(alo, ahi, si, so, b, g, be, Wn)


def _fin_body(lo_ref, hi_ref, si_ref, b_ref, g_ref, be_ref, w_ref,
              fb_ref, o_ref, st_ref):
    t = jnp.concatenate([lo_ref[...], hi_ref[...]], axis=1)
    t = t * si_ref[...] + b_ref[...]
    ph = pl.program_id(0)

    @pl.when(ph == 0)
    def _():
        _accum_stats(t, st_ref)

    @pl.when(ph == 1)
    def _():
        y = _bn_lrelu(t, st_ref, g_ref, be_ref)
        o_ref[...] = (jnp.dot(y, w_ref[...], preferred_element_type=F32)
                      + fb_ref[...])


def _fin(alo, ahi, si, b, g, be, fcW, fcb):
    return pl.pallas_call(
        _fin_body,
        grid=(2, NB),
        in_specs=[
            pl.BlockSpec((B, 16), lambda p, i: (i, 0)),
            pl.BlockSpec((B, 16), lambda p, i: (i, 0)),
            pl.BlockSpec((B, 1), lambda p, i: (i, 0)),
            pl.BlockSpec((1, 32), lambda p, i: (0, 0)),
            pl.BlockSpec((1, 32), lambda p, i: (0, 0)),
            pl.BlockSpec((1, 32), lambda p, i: (0, 0)),
            pl.BlockSpec((32, 2), lambda p, i: (0, 0)),
            pl.BlockSpec((1, 2), lambda p, i: (0, 0)),
        ],
        out_specs=pl.BlockSpec((B, 2), lambda p, i: (i, 0)),
        out_shape=jax.ShapeDtypeStruct((N, 2), F32),
        scratch_shapes=[pltpu.VMEM((8, 32), F32)],
    )(alo, ahi, si, b, g, be, fcW, fcb)


def kernel(x, edge_index, W1, b1, g1, be1, W2, b2, g2, be2, W3, b3, g3, be3,
           W4, b4, g4, be4, W5, b5, g5, be5, fcW, fcb):
    src2 = edge_index[0].reshape(ER, CH)
    dst2 = edge_index[1].reshape(ER, CH)
    cnt = _deg_kernel(src2, dst2)
    lo, hi, si, so = _pre1(x, cnt, W1)

    bs = [b1, b2, b3, b4, b5]
    gs = [g1, g2, g3, g4, g5]
    bes = [be1, be2, be3, be4, be5]
    Wn = [W2, W3, W4, W5]

    for i in range(5):
        alo, ahi = _spmm_kernel(lo, hi, src2, dst2)
        b2d = bs[i].reshape(1, 32)
        g2d = gs[i].reshape(1, 32)
        be2d = bes[i].reshape(1, 32)
        if i < 4:
            lo, hi = _mid(alo, ahi, si, so, b2d, g2d, be2d, Wn[i])
        else:
            out = _fin(alo, ahi, si, b2d, g2d, be2d, fcW,
                       fcb.reshape(1, 2))
    return out


# speedup vs baseline: 1.7021x; 1.7021x over previous
import functools

import jax
import jax.numpy as jnp
from jax import lax
from jax.experimental import pallas as pl
from jax.experimental.pallas import tpu as pltpu
from jax.experimental.pallas import tpu_sc as plsc

N = 100000
E = 1600000
NC = 2
NS = 16
CH = 100
ER = E // CH
KS = 8
RPT = ER // NS
NSTEP = RPT // KS
RB = 1000
NBLK = N // RB
ZB = 128
NZB = N // ZB
ZTAIL = N - NZB * ZB
B = 4000
NB = N // B
F32 = jnp.float32

_mesh = plsc.VectorSubcoreMesh(core_axis_name="c", subcore_axis_name="s")


def _zero_acc(zbuf, acc, s):
    zero = jnp.zeros((16,), F32)

    @pl.loop(0, ZB)
    def _(i):
        zbuf[i, :] = zero

    @pl.loop(s, NZB, step=NS)
    def _(j):
        pltpu.sync_copy(zbuf, acc.at[pl.ds(j * ZB, ZB)])

    @pl.when(s == 0)
    def _():
        pltpu.sync_copy(zbuf.at[pl.ds(0, ZTAIL)],
                        acc.at[pl.ds(NZB * ZB, ZTAIL)])


@functools.partial(
    pl.kernel,
    out_type=jax.ShapeDtypeStruct((NC, N, 16), F32),
    mesh=_mesh,
    compiler_params=pltpu.CompilerParams(use_tc_tiling_on_sc=False),
    scratch_types=[
        pltpu.VMEM((KS, CH), jnp.int32),
        pltpu.VMEM((CH, 16), F32),
        pltpu.VMEM((ZB, 16), F32),
        pltpu.VMEM_SHARED((N, 16), F32),
        pltpu.SemaphoreType.DMA,
    ],
)
def _deg_kernel(src2_hbm, dst2_hbm, out_hbm, idx, ones_r, zbuf, acc, sem):
    c = lax.axis_index("c")
    s = lax.axis_index("s")
    one = jnp.ones((16,), F32)

    @pl.loop(0, CH)
    def _(i):
        ones_r[i, :] = one

    _zero_acc(zbuf, acc, s)

    plsc.subcore_barrier()

    row0 = s * RPT

    def count_pass(e2_hbm):
        @pl.loop(0, NSTEP)
        def _(j):
            r = pl.multiple_of(row0 + j * KS, 8)
            pltpu.sync_copy(e2_hbm.at[pl.ds(r, KS)], idx)
            descs = [pltpu.async_copy(ones_r, acc.at[idx.at[k]], sem,
                                      add=True) for k in range(KS)]
            for d in descs:
                d.wait()

    @pl.when(c == 0)
    def _():
        count_pass(src2_hbm)

    @pl.when(c == 1)
    def _():
        count_pass(dst2_hbm)

    plsc.subcore_barrier()

    @pl.loop(s, NBLK, step=NS)
    def _(j):
        pltpu.sync_copy(acc.at[pl.ds(j * RB, RB)],
                        out_hbm.at[c, pl.ds(j * RB, RB)])


@functools.partial(
    pl.kernel,
    out_type=[jax.ShapeDtypeStruct((N, 16), F32),
              jax.ShapeDtypeStruct((N, 16), F32)],
    mesh=_mesh,
    compiler_params=pltpu.CompilerParams(use_tc_tiling_on_sc=False),
    scratch_types=[
        pltpu.VMEM((KS, CH), jnp.int32),
        pltpu.VMEM((KS, CH), jnp.int32),
        pltpu.VMEM((KS, CH), jnp.int32),
        pltpu.VMEM((KS, CH), jnp.int32),
        pltpu.VMEM((KS * CH, 16), F32),
        pltpu.VMEM((KS * CH, 16), F32),
        pltpu.VMEM((ZB, 16), F32),
        pltpu.VMEM_SHARED((N, 16), F32),
        pltpu.SemaphoreType.DMA,
        pltpu.SemaphoreType.DMA,
        pltpu.SemaphoreType.DMA,
        pltpu.SemaphoreType.DMA,
    ],
)
def _spmm_kernel(hlo_hbm, hhi_hbm, src2_hbm, dst2_hbm, olo_hbm, ohi_hbm,
                 sidx0, didx0, sidx1, didx1, rows0, rows1, zbuf, acc,
                 gsem0, gsem1, ssem0, ssem1):
    c = lax.axis_index("c")
    s = lax.axis_index("s")

    _zero_acc(zbuf, acc, s)

    plsc.subcore_barrier()

    row0 = s * RPT
    HB = KS * CH

    def edge_pass(h_hbm):
        def load_idx(j, si, di):
            r = pl.multiple_of(row0 + j * KS, 8)
            pltpu.sync_copy(src2_hbm.at[pl.ds(r, KS)], si)
            pltpu.sync_copy(dst2_hbm.at[pl.ds(r, KS)], di)

        def fire_g(si, rb, sem):
            for k in range(KS):
                pltpu.async_copy(h_hbm.at[si.at[k]],
                                 rb.at[pl.ds(k * CH, CH)], sem)

        def fire_s(di, rb, sem):
            for k in range(KS):
                pltpu.async_copy(rb.at[pl.ds(k * CH, CH)],
                                 acc.at[di.at[k]], sem, add=True)

        def drain(sem, rb):
            pltpu.make_async_copy(h_hbm.at[pl.ds(0, HB)], rb, sem).wait()

        load_idx(0, sidx0, didx0)
        fire_g(sidx0, rows0, gsem0)

        @pl.loop(0, NSTEP // 2)
        def _(t):
            j = 2 * t
            load_idx(j + 1, sidx1, didx1)
            fire_g(sidx1, rows1, gsem1)
            drain(gsem0, rows0)
            fire_s(didx0, rows0, ssem0)
            drain(ssem0, rows0)

            @pl.when(t + 1 < NSTEP // 2)
            def _():
                load_idx(j + 2, sidx0, didx0)
                fire_g(sidx0, rows0, gsem0)

            drain(gsem1, rows1)
            fire_s(didx1, rows1, ssem1)
            drain(ssem1, rows1)

        if NSTEP % 2 == 1:
            load_idx(NSTEP - 1, sidx0, didx0)
            fire_g(sidx0, rows0, gsem0)
            drain(gsem0, rows0)
            fire_s(didx0, rows0, ssem0)
            drain(ssem0, rows0)

    @pl.when(c == 0)
    def _():
        edge_pass(hlo_hbm)

    @pl.when(c == 1)
    def _():
        edge_pass(hhi_hbm)

    plsc.subcore_barrier()

    def copy_out(o_hbm):
        @pl.loop(s, NBLK, step=NS)
        def _(j):
            pltpu.sync_copy(acc.at[pl.ds(j * RB, RB)],
                            o_hbm.at[pl.ds(j * RB, RB)])

    @pl.when(c == 0)
    def _():
        copy_out(olo_hbm)

    @pl.when(c == 1)
    def _():
        copy_out(ohi_hbm)


def _pre1_body(x_ref, cnt_ref, w_ref, lo_ref, hi_ref):
    so = lax.rsqrt(jnp.maximum(cnt_ref[0, :, 0:1], 1.0))
    z = jnp.dot(x_ref[...] * so, w_ref[...], preferred_element_type=F32)
    lo_ref[...] = z[:, :16]
    hi_ref[...] = z[:, 16:]


def _pre1(x, cnt, W1):
    return pl.pallas_call(
        _pre1_body,
        grid=(NB,),
        in_specs=[
            pl.BlockSpec((B, 33), lambda i: (i, 0)),
            pl.BlockSpec((NC, B, 16), lambda i: (0, i, 0)),
            pl.BlockSpec((33, 32), lambda i: (0, 0)),
        ],
        out_specs=[
            pl.BlockSpec((B, 16), lambda i: (i, 0)),
            pl.BlockSpec((B, 16), lambda i: (i, 0)),
        ],
        out_shape=[
            jax.ShapeDtypeStruct((N, 16), F32),
            jax.ShapeDtypeStruct((N, 16), F32),
        ],
    )(x, cnt, W1)


P = N // 8


def _scales_body(cntp_ref, sip_ref, sop_ref):
    sop_ref[...] = lax.rsqrt(jnp.maximum(cntp_ref[0], 1.0))
    sip_ref[...] = lax.rsqrt(jnp.maximum(cntp_ref[1], 1.0))


def _scales(cnt_p):
    return pl.pallas_call(
        _scales_body,
        out_shape=[jax.ShapeDtypeStruct((P, 128), F32),
                   jax.ShapeDtypeStruct((P, 128), F32)],
    )(cnt_p)


def _statsp_body(alo_ref, ahi_ref, sip_ref, blo_ref, bhi_ref, st_ref):
    si = sip_ref[...]
    tlo = alo_ref[...] * si + blo_ref[...]
    thi = ahi_ref[...] * si + bhi_ref[...]
    st_ref[...] = jnp.concatenate([
        jnp.sum(tlo, axis=0, keepdims=True),
        jnp.sum(tlo * tlo, axis=0, keepdims=True),
        jnp.sum(thi, axis=0, keepdims=True),
        jnp.sum(thi * thi, axis=0, keepdims=True),
        jnp.zeros((4, 128), F32)], axis=0)


def _statsp(alo_p, ahi_p, si_p, blo, bhi):
    return pl.pallas_call(
        _statsp_body,
        out_shape=jax.ShapeDtypeStruct((8, 128), F32),
    )(alo_p, ahi_p, si_p, blo, bhi)


def _bn128(t, st_row_s1, st_row_s2, fold_ref, spread_ref, g_ref, be_ref):
    s1 = jnp.dot(st_row_s1, fold_ref[...], preferred_element_type=F32) / N
    s2 = jnp.dot(st_row_s2, fold_ref[...], preferred_element_type=F32) / N
    var = s2 - s1 * s1
    rs = lax.rsqrt(var + 1e-5)
    mu128 = jnp.dot(s1, spread_ref[...], preferred_element_type=F32)
    rs128 = jnp.dot(rs, spread_ref[...], preferred_element_type=F32)
    y = (t - mu128) * rs128 * g_ref[...] + be_ref[...]
    return jnp.where(y >= 0, y, 0.01 * y)


def _applyp_body(alo_ref, ahi_ref, sip_ref, sop_ref, st_ref, blo_ref,
                 bhi_ref, glo_ref, ghi_ref, belo_ref, behi_ref, fold_ref,
                 spread_ref, dll_ref, dlh_ref, dhl_ref, dhh_ref,
                 olo_ref, ohi_ref):
    si = sip_ref[...]
    so = sop_ref[...]
    tlo = alo_ref[...] * si + blo_ref[...]
    thi = ahi_ref[...] * si + bhi_ref[...]
    ylo = _bn128(tlo, st_ref[0:1, :], st_ref[1:2, :], fold_ref, spread_ref,
                 glo_ref, belo_ref) * so
    yhi = _bn128(thi, st_ref[2:3, :], st_ref[3:4, :], fold_ref, spread_ref,
                 ghi_ref, behi_ref) * so
    olo_ref[...] = (jnp.dot(ylo, dll_ref[...], preferred_element_type=F32)
                    + jnp.dot(yhi, dhl_ref[...], preferred_element_type=F32))
    ohi_ref[...] = (jnp.dot(ylo, dlh_ref[...], preferred_element_type=F32)
                    + jnp.dot(yhi, dhh_ref[...], preferred_element_type=F32))


def _applyp(alo_p, ahi_p, si_p, so_p, st, blo, bhi, glo, ghi, belo, behi,
            fold, spread, dll, dlh, dhl, dhh):
    return pl.pallas_call(
        _applyp_body,
        out_shape=[jax.ShapeDtypeStruct((P, 128), F32),
                   jax.ShapeDtypeStruct((P, 128), F32)],
    )(alo_p, ahi_p, si_p, so_p, st, blo, bhi, glo, ghi, belo, behi,
      fold, spread, dll, dlh, dhl, dhh)


def _finp_body(alo_ref, ahi_ref, sip_ref, st_ref, blo_ref, bhi_ref,
               glo_ref, ghi_ref, belo_ref, behi_ref, fold_ref, spread_ref,
               dfl_ref, dfh_ref, fb_ref, o_ref):
    si = sip_ref[...]
    tlo = alo_ref[...] * si + blo_ref[...]
    thi = ahi_ref[...] * si + bhi_ref[...]
    ylo = _bn128(tlo, st_ref[0:1, :], st_ref[1:2, :], fold_ref, spread_ref,
                 glo_ref, belo_ref)
    yhi = _bn128(thi, st_ref[2:3, :], st_ref[3:4, :], fold_ref, spread_ref,
                 ghi_ref, behi_ref)
    o_ref[...] = (jnp.dot(ylo, dfl_ref[...], preferred_element_type=F32)
                  + jnp.dot(yhi, dfh_ref[...], preferred_element_type=F32)
                  + fb_ref[...])


def _finp(alo_p, ahi_p, si_p, st, blo, bhi, glo, ghi, belo, behi,
          fold, spread, dfl, dfh, fb16):
    return pl.pallas_call(
        _finp_body,
        out_shape=jax.ShapeDtypeStruct((P, 16), F32),
    )(alo_p, ahi_p, si_p, st, blo, bhi, glo, ghi, belo, behi,
      fold, spread, dfl, dfh, fb16)


def _tile8(v):
    return jnp.tile(v, 8).reshape(1, -1)


def kernel(x, edge_index, W1, b1, g1, be1, W2, b2, g2, be2, W3, b3, g3, be3,
           W4, b4, g4, be4, W5, b5, g5, be5, fcW, fcb):
    src2 = edge_index[0].reshape(ER, CH)
    dst2 = edge_index[1].reshape(ER, CH)
    cnt = _deg_kernel(src2, dst2)
    si_p, so_p = _scales(cnt.reshape(NC, P, 128))
    lo, hi = _pre1(x, cnt, W1)

    eye8 = jnp.eye(8, dtype=F32)
    fold = jnp.tile(jnp.eye(16, dtype=F32), (8, 1))
    spread = fold.T
    bs = [b1, b2, b3, b4, b5]
    gs = [g1, g2, g3, g4, g5]
    bes = [be1, be2, be3, be4, be5]
    Wn = [W2, W3, W4, W5]

    for i in range(5):
        alo, ahi = _spmm_kernel(lo, hi, src2, dst2)
        alo_p = alo.reshape(P, 128)
        ahi_p = ahi.reshape(P, 128)
        blo, bhi = _tile8(bs[i][:16]), _tile8(bs[i][16:])
        glo, ghi = _tile8(gs[i][:16]), _tile8(gs[i][16:])
        belo, behi = _tile8(bes[i][:16]), _tile8(bes[i][16:])
        st = _statsp(alo_p, ahi_p, si_p, blo, bhi)
        if i < 4:
            W = Wn[i]
            lo_p, hi_p = _applyp(
                alo_p, ahi_p, si_p, so_p, st, blo, bhi, glo, ghi, belo,
                behi, fold, spread,
                jnp.kron(eye8, W[:16, :16]), jnp.kron(eye8, W[:16, 16:]),
                jnp.kron(eye8, W[16:, :16]), jnp.kron(eye8, W[16:, 16:]))
            lo = lo_p.reshape(N, 16)
            hi = hi_p.reshape(N, 16)
        else:
            out_p = _finp(
                alo_p, ahi_p, si_p, st, blo, bhi, glo, ghi, belo, behi,
                fold, spread,
                jnp.kron(eye8, fcW[:16, :]), jnp.kron(eye8, fcW[16:, :]),
                _tile8(fcb))
            out = out_p.reshape(N, 2)
    return out
